# baseline (speedup 1.0000x reference)
import functools

import jax
import jax.numpy as jnp
from jax import lax
from jax.experimental import pallas as pl
from jax.experimental.pallas import tpu as pltpu
from jax.experimental.pallas import tpu_sc as plsc

VOCAB = 1000
VPAD = 1024
EMB = 32
T = 8
BATCH = 16384

NC = 2
NS = 16
NW = NC * NS
ROWS = BATCH * T
BPW = ROWS // NW
CHUNK = 64
NCH = BPW // CHUNK


def _build_body(tok_ref, pos_ref, w_ref, b_ref, out_ref):
    t = pl.program_id(0)
    tf = jnp.dot(tok_ref[...], w_ref[...], preferred_element_type=jnp.float32)
    posw8 = jnp.dot(pos_ref[...], w_ref[...], preferred_element_type=jnp.float32)
    sel = (lax.broadcasted_iota(jnp.int32, (1, T), 1) == t).astype(jnp.float32)
    posw = jnp.dot(sel, posw8, preferred_element_type=jnp.float32)
    out_ref[...] = (tf + posw + b_ref[...])[None]


def _build_table(tok_table, pos_table, W, b):
    return pl.pallas_call(
        _build_body,
        grid=(T,),
        in_specs=[
            pl.BlockSpec((VOCAB, EMB), lambda t: (0, 0)),
            pl.BlockSpec((T, EMB), lambda t: (0, 0)),
            pl.BlockSpec((EMB, VOCAB), lambda t: (0, 0)),
            pl.BlockSpec((1, VOCAB), lambda t: (0, 0)),
        ],
        out_specs=pl.BlockSpec((1, VOCAB, VOCAB), lambda t: (t, 0, 0)),
        out_shape=jax.ShapeDtypeStruct((T, VOCAB, VOCAB), jnp.float32),
    )(tok_table, pos_table, W, b.reshape(1, VOCAB))


_sc_mesh = plsc.VectorSubcoreMesh(core_axis_name="c", subcore_axis_name="s")


@functools.partial(
    pl.kernel,
    mesh=_sc_mesh,
    out_type=jax.ShapeDtypeStruct((ROWS, VOCAB), jnp.float32),
    scratch_types=[
        pltpu.VMEM((BPW,), jnp.int32),
        pltpu.VMEM((CHUNK, VOCAB), jnp.float32),
        pltpu.SemaphoreType.DMA,
    ],
    compiler_params=pltpu.CompilerParams(use_tc_tiling_on_sc=False),
)
def _sc_gather(table_hbm, idx_hbm, out_hbm, cidx_v, rows_v, sem):
    wid = lax.axis_index("s") * NC + lax.axis_index("c")
    base = wid * BPW
    pltpu.sync_copy(idx_hbm.at[pl.ds(base, BPW)], cidx_v)
    tvec = (lax.broadcasted_iota(jnp.int32, (16,), 0) & (T - 1)) * VOCAB

    def add_body(k, carry):
        sl = pl.ds(k * 16, 16)
        cidx_v[sl] = cidx_v[sl] + tvec
        return carry

    lax.fori_loop(0, BPW // 16, add_body, 0)

    def chunk_body(c, carry):
        pltpu.async_copy(
            table_hbm.at[cidx_v.at[pl.ds(c * CHUNK, CHUNK)]], rows_v, sem
        ).wait()
        pltpu.sync_copy(rows_v, out_hbm.at[pl.ds(base + c * CHUNK, CHUNK)])
        return carry

    lax.fori_loop(0, NCH, chunk_body, 0)


def kernel(idx, tok_table, pos_table, W, b):
    table = _build_table(tok_table, pos_table, W, b)
    idx_flat = idx.reshape(ROWS).astype(jnp.int32)
    out = _sc_gather(table.reshape(T * VOCAB, VOCAB), idx_flat)
    return out.reshape(BATCH, T, VOCAB)


# speedup vs baseline: 1.6419x; 1.6419x over previous
import functools

import jax
import jax.numpy as jnp
from jax import lax
from jax.experimental import pallas as pl
from jax.experimental.pallas import tpu as pltpu
from jax.experimental.pallas import tpu_sc as plsc

VOCAB = 1000
EMB = 32
EPAD = 128
T = 8
BATCH = 16384

NC = 2
NS = 16
NW = NC * NS
ROWS = BATCH * T
BPW = ROWS // NW
CHUNK = 128
NCH = BPW // CHUNK

_sc_mesh = plsc.VectorSubcoreMesh(core_axis_name="c", subcore_axis_name="s")


@functools.partial(
    pl.kernel,
    mesh=_sc_mesh,
    out_type=jax.ShapeDtypeStruct((ROWS, EPAD), jnp.float32),
    scratch_types=[
        pltpu.VMEM((BPW,), jnp.int32),
        pltpu.VMEM((CHUNK, EPAD), jnp.float32),
        pltpu.VMEM((CHUNK, EPAD), jnp.float32),
        pltpu.SemaphoreType.DMA,
        pltpu.SemaphoreType.DMA,
        pltpu.SemaphoreType.DMA,
        pltpu.SemaphoreType.DMA,
    ],
)
def _sc_gather(tok_hbm, idx_hbm, out_hbm, idx_v, rows0, rows1, g0, g1, w0, w1):
    wid = lax.axis_index("s") * NC + lax.axis_index("c")
    base = wid * BPW
    pltpu.sync_copy(idx_hbm.at[pl.ds(base, BPW)], idx_v)

    rows = (rows0, rows1)
    gsem = (g0, g1)
    wsem = (w0, w1)

    def g_copy(c, b):
        return pltpu.make_async_copy(
            tok_hbm.at[idx_v.at[pl.ds(c * CHUNK, CHUNK)]], rows[b], gsem[b]
        )

    def w_copy(c, b):
        return pltpu.make_async_copy(
            rows[b], out_hbm.at[pl.ds(base + c * CHUNK, CHUNK)], wsem[b]
        )

    g_copy(0, 0).start()
    g_copy(0, 0).wait()
    w_copy(0, 0).start()
    g_copy(1, 1).start()

    for c in range(1, NCH - 1):
        g_copy(c, c & 1).wait()
        w_copy(c, c & 1).start()
        w_copy(c - 1, (c - 1) & 1).wait()
        g_copy(c + 1, (c + 1) & 1).start()

    c = NCH - 1
    g_copy(c, c & 1).wait()
    w_copy(c, c & 1).start()
    w_copy(c - 1, (c - 1) & 1).wait()
    w_copy(c, c & 1).wait()


def _proj_body(x_ref, pos_ref, w_ref, b_ref, out_ref):
    r = x_ref.shape[0]
    x = x_ref[...] + pos_ref[...][None]
    logits = jnp.dot(
        x.reshape(r * T, EPAD), w_ref[...], preferred_element_type=jnp.float32
    )
    out_ref[...] = logits.reshape(r, T, VOCAB) + b_ref[...][None]


RB = 64


def _project(x3, pos_p, Wp, b):
    return pl.pallas_call(
        _proj_body,
        grid=(BATCH // RB,),
        in_specs=[
            pl.BlockSpec((RB, T, EPAD), lambda i: (i, 0, 0)),
            pl.BlockSpec((T, EPAD), lambda i: (0, 0)),
            pl.BlockSpec((EPAD, VOCAB), lambda i: (0, 0)),
            pl.BlockSpec((1, VOCAB), lambda i: (0, 0)),
        ],
        out_specs=pl.BlockSpec((RB, T, VOCAB), lambda i: (i, 0, 0)),
        out_shape=jax.ShapeDtypeStruct((BATCH, T, VOCAB), jnp.float32),
        compiler_params=pltpu.CompilerParams(
            dimension_semantics=("arbitrary",),
        ),
    )(x3, pos_p, Wp, b.reshape(1, VOCAB))


def kernel(idx, tok_table, pos_table, W, b):
    idx_flat = idx.reshape(ROWS).astype(jnp.int32)
    tok_p = jnp.pad(tok_table, ((0, 0), (0, EPAD - EMB)))
    pos_p = jnp.pad(pos_table, ((0, 0), (0, EPAD - EMB)))
    Wp = jnp.pad(W, ((0, EPAD - EMB), (0, 0)))
    x = _sc_gather(tok_p, idx_flat)
    return _project(x.reshape(BATCH, T, EPAD), pos_p, Wp, b)
